# baseline (speedup 1.0000x reference)
import functools

import jax
import jax.numpy as jnp
from jax import lax
from jax.experimental import pallas as pl
from jax.experimental.pallas import tpu as pltpu
from jax.experimental.pallas import tpu_sc as plsc

B = 16384
DIM = 64
NW = 32
ROWS_PER_W = B // NW
GROUPS = ROWS_PER_W // 16


def _body(mT_hbm, gT_hbm, midx_hbm, gidx_hbm, wb_hbm, out_hbm,
          midx_v, gidx_v, mcols_v, gcols_v, res_v, wb_v, sem):
    wid = lax.axis_index("s") * 2 + lax.axis_index("c")
    base = wid * ROWS_PER_W

    pltpu.sync_copy(midx_hbm.at[pl.ds(base, ROWS_PER_W)], midx_v)
    pltpu.sync_copy(gidx_hbm.at[pl.ds(base, ROWS_PER_W)], gidx_v)
    pltpu.sync_copy(wb_hbm, wb_v)

    def fire(j, _):
        pltpu.async_copy(mT_hbm.at[j].at[midx_v], mcols_v.at[j], sem)
        pltpu.async_copy(gT_hbm.at[j].at[gidx_v], gcols_v.at[j], sem)
        return 0

    lax.fori_loop(0, DIM, fire, 0)

    pltpu.make_async_copy(mT_hbm.at[:, pl.ds(0, ROWS_PER_W)], mcols_v, sem).wait()
    pltpu.make_async_copy(gT_hbm.at[:, pl.ds(0, ROWS_PER_W)], gcols_v, sem).wait()

    w = wb_v[0, :]
    bb = wb_v[1, :]

    def group(gi, _):
        off = gi * 16
        zero = jnp.zeros((16,), jnp.float32)
        dot = zero
        nm2 = zero
        ng2 = zero
        for j in range(DIM):
            mj = mcols_v[j, pl.ds(off, 16)]
            gj = gcols_v[j, pl.ds(off, 16)]
            dot = dot + mj * gj
            nm2 = nm2 + mj * mj
            ng2 = ng2 + gj * gj
        d = jnp.maximum(nm2, 1e-16) * jnp.maximum(ng2, 1e-16)
        di = plsc.bitcast(d, jnp.int32)
        y = plsc.bitcast(jnp.int32(0x5F3759DF) - (di >> 1), jnp.float32)
        for _ in range(3):
            y = y * (1.5 - 0.5 * d * y * y)
        cos = dot * y
        z = cos * w + bb
        sig = 1.0 / (1.0 + jnp.exp(-z))
        res_v[pl.ds(off, 16)] = sig
        return 0

    lax.fori_loop(0, GROUPS, group, 0)

    pltpu.sync_copy(res_v, out_hbm.at[pl.ds(base, ROWS_PER_W)])


@jax.jit
def _run(mT, gT, midx, gidx, wb):
    mesh = plsc.VectorSubcoreMesh(core_axis_name="c", subcore_axis_name="s")
    f = functools.partial(
        pl.kernel,
        mesh=mesh,
        out_type=jax.ShapeDtypeStruct((B,), jnp.float32),
        scratch_types=[
            pltpu.VMEM((ROWS_PER_W,), jnp.int32),
            pltpu.VMEM((ROWS_PER_W,), jnp.int32),
            pltpu.VMEM((DIM, ROWS_PER_W), jnp.float32),
            pltpu.VMEM((DIM, ROWS_PER_W), jnp.float32),
            pltpu.VMEM((ROWS_PER_W,), jnp.float32),
            pltpu.VMEM((2, 16), jnp.float32),
            pltpu.SemaphoreType.DMA,
        ],
        compiler_params=pltpu.CompilerParams(
            needs_layout_passes=False, use_tc_tiling_on_sc=False
        ),
    )(_body)
    return f(mT, gT, midx, gidx, wb)


def kernel(x, m_table, g_table, fc_w, fc_b):
    mT = m_table.T
    gT = g_table.T
    midx = x[:, 0].astype(jnp.int32)
    gidx = x[:, 1].astype(jnp.int32)
    wb = jnp.stack([
        jnp.broadcast_to(fc_w.reshape(()), (16,)),
        jnp.broadcast_to(fc_b.reshape(()), (16,)),
    ]).astype(jnp.float32)
    out = _run(mT, gT, midx, gidx, wb)
    return out.reshape(B, 1)


# speedup vs baseline: 12.8965x; 12.8965x over previous
import functools

import jax
import jax.numpy as jnp
from jax import lax
from jax.experimental import pallas as pl
from jax.experimental.pallas import tpu as pltpu
from jax.experimental.pallas import tpu_sc as plsc

B = 16384
DIM = 64
NW = 32
ROWS_PER_W = B // NW
CHUNK = 256


def _body(m_hbm, g_hbm, midx_hbm, gidx_hbm, wb_hbm, out_hbm,
          midx_v, gidx_v, mrows_v, grows_v, res_v, wb_v, sem):
    wid = lax.axis_index("s") * 2 + lax.axis_index("c")
    base = wid * ROWS_PER_W

    pltpu.sync_copy(midx_hbm.at[pl.ds(base, ROWS_PER_W)], midx_v)
    pltpu.sync_copy(gidx_hbm.at[pl.ds(base, ROWS_PER_W)], gidx_v)
    pltpu.sync_copy(wb_hbm, wb_v)

    w = wb_v[0, :]
    bb = wb_v[1, :]
    lane = lax.iota(jnp.int32, 16)

    def chunk_body(c, _):
        coff = c * CHUNK

        def fire(g, _):
            mv = midx_v[pl.ds(coff + g * 16, 16)]
            gv = gidx_v[pl.ds(coff + g * 16, 16)]
            for k in range(16):
                pltpu.async_copy(m_hbm.at[pl.ds(mv[k], 1)],
                                 mrows_v.at[pl.ds(g * 16 + k, 1)], sem)
                pltpu.async_copy(g_hbm.at[pl.ds(gv[k], 1)],
                                 grows_v.at[pl.ds(g * 16 + k, 1)], sem)
            return 0

        lax.fori_loop(0, CHUNK // 16, fire, 0)

        pltpu.make_async_copy(m_hbm.at[pl.ds(0, CHUNK)], mrows_v, sem).wait()
        pltpu.make_async_copy(g_hbm.at[pl.ds(0, CHUNK)], grows_v, sem).wait()

        def group(gi, _):
            rowv = gi * 16 + lane
            zero = jnp.zeros((16,), jnp.float32)
            dot = zero
            nm2 = zero
            ng2 = zero
            for j in range(DIM):
                colv = jnp.full((16,), j, jnp.int32)
                mj = plsc.load_gather(mrows_v, [rowv, colv])
                gj = plsc.load_gather(grows_v, [rowv, colv])
                dot = dot + mj * gj
                nm2 = nm2 + mj * mj
                ng2 = ng2 + gj * gj
            d = jnp.maximum(nm2, 1e-16) * jnp.maximum(ng2, 1e-16)
            di = plsc.bitcast(d, jnp.int32)
            y = plsc.bitcast(jnp.int32(0x5F3759DF) - (di >> 1), jnp.float32)
            for _ in range(3):
                y = y * (1.5 - 0.5 * d * y * y)
            cos = dot * y
            z = cos * w + bb
            sig = 1.0 / (1.0 + jnp.exp(-z))
            res_v[pl.ds(coff + gi * 16, 16)] = sig
            return 0

        lax.fori_loop(0, CHUNK // 16, group, 0)
        return 0

    lax.fori_loop(0, ROWS_PER_W // CHUNK, chunk_body, 0)

    pltpu.sync_copy(res_v, out_hbm.at[pl.ds(base, ROWS_PER_W)])


@jax.jit
def _run(m_table, g_table, midx, gidx, wb):
    mesh = plsc.VectorSubcoreMesh(core_axis_name="c", subcore_axis_name="s")
    f = functools.partial(
        pl.kernel,
        mesh=mesh,
        out_type=jax.ShapeDtypeStruct((B,), jnp.float32),
        scratch_types=[
            pltpu.VMEM((ROWS_PER_W,), jnp.int32),
            pltpu.VMEM((ROWS_PER_W,), jnp.int32),
            pltpu.VMEM((CHUNK, DIM), jnp.float32),
            pltpu.VMEM((CHUNK, DIM), jnp.float32),
            pltpu.VMEM((ROWS_PER_W,), jnp.float32),
            pltpu.VMEM((2, 16), jnp.float32),
            pltpu.SemaphoreType.DMA,
        ],
        compiler_params=pltpu.CompilerParams(needs_layout_passes=False),
    )(_body)
    return f(m_table, g_table, midx, gidx, wb)


def kernel(x, m_table, g_table, fc_w, fc_b):
    midx = x[:, 0].astype(jnp.int32)
    gidx = x[:, 1].astype(jnp.int32)
    wb = jnp.stack([
        jnp.broadcast_to(fc_w.reshape(()), (16,)),
        jnp.broadcast_to(fc_b.reshape(()), (16,)),
    ]).astype(jnp.float32)
    out = _run(m_table, g_table, midx, gidx, wb)
    return out.reshape(B, 1)


# speedup vs baseline: 20.2154x; 1.5675x over previous
import functools

import jax
import jax.numpy as jnp
from jax import lax
from jax.experimental import pallas as pl
from jax.experimental.pallas import tpu as pltpu
from jax.experimental.pallas import tpu_sc as plsc

B = 16384
DIM = 64
NW = 32
ROWS_PER_W = B // NW
GS = 8
GROUPS = ROWS_PER_W // GS
PAD = 16


def _body(mT_hbm, g_hbm, midx_hbm, gidx_hbm, wb_hbm, out_hbm,
          midx_v, gidx_v, winbuf_v, grows_v, res_v, tmp_v, wb_v, sem):
    wid = lax.axis_index("s") * 2 + lax.axis_index("c")
    base = wid * ROWS_PER_W

    pltpu.sync_copy(midx_hbm.at[pl.ds(base, ROWS_PER_W)],
                    midx_v.at[pl.ds(0, ROWS_PER_W)])
    pltpu.sync_copy(gidx_hbm.at[pl.ds(base, ROWS_PER_W)],
                    gidx_v.at[pl.ds(0, ROWS_PER_W)])
    pltpu.sync_copy(wb_hbm, wb_v)

    w = wb_v[0, :]
    bb = wb_v[1, :]
    iota = lax.iota(jnp.int32, 16)
    samp16 = iota & 7
    jhalf = iota >> 3
    fold_idx = iota ^ 8
    mask8 = iota < 8

    def group(g, _):
        goff = g * GS
        mv = midx_v[pl.ds(goff, 16)]
        gv = gidx_v[pl.ds(goff, 16)]
        for k in range(GS):
            woff = pl.multiple_of((mv[k] >> 7) * 128, 128)
            pltpu.async_copy(mT_hbm.at[:, pl.ds(woff, 128)],
                             winbuf_v.at[k], sem)
            pltpu.async_copy(g_hbm.at[pl.ds(gv[k], 1)],
                             grows_v.at[pl.ds(k, 1)], sem)
        for k in range(GS):
            pltpu.make_async_copy(mT_hbm.at[:, pl.ds(0, 128)],
                                  winbuf_v.at[k], sem).wait()
        pltpu.make_async_copy(g_hbm.at[pl.ds(0, GS)], grows_v, sem).wait()

        lanevec = plsc.load_gather(midx_v, [goff + samp16]) & 127
        zero = jnp.zeros((16,), jnp.float32)
        dot = zero
        nm2 = zero
        ng2 = zero
        for jb in range(0, DIM, 2):
            jv = jhalf + jb
            mj = plsc.load_gather(winbuf_v, [samp16, jv, lanevec])
            gj = plsc.load_gather(grows_v, [samp16, jv])
            dot = dot + mj * gj
            nm2 = nm2 + mj * mj
            ng2 = ng2 + gj * gj
        tmp_v[0, :] = dot
        tmp_v[1, :] = nm2
        tmp_v[2, :] = ng2
        dot = dot + plsc.load_gather(tmp_v, [jnp.zeros((16,), jnp.int32),
                                             fold_idx])
        nm2 = nm2 + plsc.load_gather(tmp_v, [jnp.ones((16,), jnp.int32),
                                             fold_idx])
        ng2 = ng2 + plsc.load_gather(tmp_v, [jnp.full((16,), 2, jnp.int32),
                                             fold_idx])
        d = jnp.maximum(nm2, 1e-16) * jnp.maximum(ng2, 1e-16)
        di = plsc.bitcast(d, jnp.int32)
        y = plsc.bitcast(jnp.int32(0x5F3759DF) - (di >> 1), jnp.float32)
        for _ in range(3):
            y = y * (1.5 - 0.5 * d * y * y)
        cos = dot * y
        z = cos * w + bb
        sig = 1.0 / (1.0 + jnp.exp(-z))
        plsc.store_compressed(res_v.at[pl.ds(goff, 16)], sig, mask=mask8)
        return 0

    lax.fori_loop(0, GROUPS, group, 0)

    pltpu.sync_copy(res_v.at[pl.ds(0, ROWS_PER_W)],
                    out_hbm.at[pl.ds(base, ROWS_PER_W)])


@jax.jit
def _run(mT, g_table, midx, gidx, wb):
    mesh = plsc.VectorSubcoreMesh(core_axis_name="c", subcore_axis_name="s")
    f = functools.partial(
        pl.kernel,
        mesh=mesh,
        out_type=jax.ShapeDtypeStruct((B,), jnp.float32),
        scratch_types=[
            pltpu.VMEM((ROWS_PER_W + PAD,), jnp.int32),
            pltpu.VMEM((ROWS_PER_W + PAD,), jnp.int32),
            pltpu.VMEM((GS, DIM, 128), jnp.float32),
            pltpu.VMEM((GS, DIM), jnp.float32),
            pltpu.VMEM((ROWS_PER_W + PAD,), jnp.float32),
            pltpu.VMEM((3, 16), jnp.float32),
            pltpu.VMEM((2, 16), jnp.float32),
            pltpu.SemaphoreType.DMA,
        ],
        compiler_params=pltpu.CompilerParams(needs_layout_passes=False),
    )(_body)
    return f(mT, g_table, midx, gidx, wb)


def kernel(x, m_table, g_table, fc_w, fc_b):
    mT = m_table.T
    midx = x[:, 0].astype(jnp.int32)
    gidx = x[:, 1].astype(jnp.int32)
    wb = jnp.stack([
        jnp.broadcast_to(fc_w.reshape(()), (16,)),
        jnp.broadcast_to(fc_b.reshape(()), (16,)),
    ]).astype(jnp.float32)
    out = _run(mT, g_table, midx, gidx, wb)
    return out.reshape(B, 1)


# speedup vs baseline: 20.7239x; 1.0252x over previous
import functools

import jax
import jax.numpy as jnp
from jax import lax
from jax.experimental import pallas as pl
from jax.experimental.pallas import tpu as pltpu
from jax.experimental.pallas import tpu_sc as plsc

B = 16384
DIM = 64
NW = 32
ROWS_PER_W = B // NW
GS = 4
GROUPS = ROWS_PER_W // GS
PAD = 16


def _body(mT_hbm, g_hbm, midx_hbm, gidx_hbm, wb_hbm, out_hbm,
          midx_v, gidx_v, win0_v, win1_v, gr0_v, gr1_v, res_v, tmp_v, wb_v,
          sem0, sem1):
    wid = lax.axis_index("s") * 2 + lax.axis_index("c")
    base = wid * ROWS_PER_W

    pltpu.sync_copy(midx_hbm.at[pl.ds(base, ROWS_PER_W)],
                    midx_v.at[pl.ds(0, ROWS_PER_W)])
    pltpu.sync_copy(gidx_hbm.at[pl.ds(base, ROWS_PER_W)],
                    gidx_v.at[pl.ds(0, ROWS_PER_W)])
    pltpu.sync_copy(wb_hbm, wb_v)

    w = wb_v[0, :]
    bb = wb_v[1, :]
    iota = lax.iota(jnp.int32, 16)
    samp16 = iota & 3
    jq = iota >> 2
    fold8 = iota ^ 8
    fold4 = iota ^ 4
    mask4 = iota < 4
    zero16 = jnp.zeros((16,), jnp.int32)
    one16 = jnp.ones((16,), jnp.int32)
    two16 = jnp.full((16,), 2, jnp.int32)

    def fire(g, win_v, gr_v, sem):
        goff = g * GS
        mv = midx_v[pl.ds(goff, 16)]
        gv = gidx_v[pl.ds(goff, 16)]
        for k in range(GS):
            woff = pl.multiple_of((mv[k] >> 7) * 128, 128)
            pltpu.async_copy(mT_hbm.at[:, pl.ds(woff, 128)],
                             win_v.at[k], sem)
            pltpu.async_copy(g_hbm.at[pl.ds(gv[k], 1)],
                             gr_v.at[pl.ds(k, 1)], sem)

    def drain(win_v, gr_v, sem):
        for k in range(GS):
            pltpu.make_async_copy(mT_hbm.at[:, pl.ds(0, 128)],
                                  win_v.at[k], sem).wait()
        pltpu.make_async_copy(g_hbm.at[pl.ds(0, GS)], gr_v, sem).wait()

    def compute(g, win_v, gr_v):
        goff = g * GS
        lanevec = plsc.load_gather(midx_v, [goff + samp16]) & 127
        zero = jnp.zeros((16,), jnp.float32)
        dot = zero
        nm2 = zero
        ng2 = zero
        for jb in range(0, DIM, 4):
            jv = jq + jb
            mj = plsc.load_gather(win_v, [samp16, jv, lanevec])
            gj = plsc.load_gather(gr_v, [samp16, jv])
            dot = dot + mj * gj
            nm2 = nm2 + mj * mj
            ng2 = ng2 + gj * gj
        for fold in (fold8, fold4):
            tmp_v[0, :] = dot
            tmp_v[1, :] = nm2
            tmp_v[2, :] = ng2
            dot = dot + plsc.load_gather(tmp_v, [zero16, fold])
            nm2 = nm2 + plsc.load_gather(tmp_v, [one16, fold])
            ng2 = ng2 + plsc.load_gather(tmp_v, [two16, fold])
        d = jnp.maximum(nm2, 1e-16) * jnp.maximum(ng2, 1e-16)
        di = plsc.bitcast(d, jnp.int32)
        y = plsc.bitcast(jnp.int32(0x5F3759DF) - (di >> 1), jnp.float32)
        for _ in range(3):
            y = y * (1.5 - 0.5 * d * y * y)
        cos = dot * y
        z = cos * w + bb
        sig = 1.0 / (1.0 + jnp.exp(-z))
        plsc.store_compressed(res_v.at[pl.ds(goff, 16)], sig, mask=mask4)

    fire(0, win0_v, gr0_v, sem0)

    def pair(g2, _):
        g0 = g2 * 2
        fire(g0 + 1, win1_v, gr1_v, sem1)
        drain(win0_v, gr0_v, sem0)
        compute(g0, win0_v, gr0_v)

        @pl.when(g0 + 2 < GROUPS)
        def _():
            fire(g0 + 2, win0_v, gr0_v, sem0)

        drain(win1_v, gr1_v, sem1)
        compute(g0 + 1, win1_v, gr1_v)
        return 0

    lax.fori_loop(0, GROUPS // 2, pair, 0)

    pltpu.sync_copy(res_v.at[pl.ds(0, ROWS_PER_W)],
                    out_hbm.at[pl.ds(base, ROWS_PER_W)])


@jax.jit
def _run(mT, g_table, midx, gidx, wb):
    mesh = plsc.VectorSubcoreMesh(core_axis_name="c", subcore_axis_name="s")
    f = functools.partial(
        pl.kernel,
        mesh=mesh,
        out_type=jax.ShapeDtypeStruct((B,), jnp.float32),
        scratch_types=[
            pltpu.VMEM((ROWS_PER_W + PAD,), jnp.int32),
            pltpu.VMEM((ROWS_PER_W + PAD,), jnp.int32),
            pltpu.VMEM((GS, DIM, 128), jnp.float32),
            pltpu.VMEM((GS, DIM, 128), jnp.float32),
            pltpu.VMEM((GS, DIM), jnp.float32),
            pltpu.VMEM((GS, DIM), jnp.float32),
            pltpu.VMEM((ROWS_PER_W + PAD,), jnp.float32),
            pltpu.VMEM((3, 16), jnp.float32),
            pltpu.VMEM((2, 16), jnp.float32),
            pltpu.SemaphoreType.DMA,
            pltpu.SemaphoreType.DMA,
        ],
        compiler_params=pltpu.CompilerParams(needs_layout_passes=False),
    )(_body)
    return f(mT, g_table, midx, gidx, wb)


def kernel(x, m_table, g_table, fc_w, fc_b):
    mT = m_table.T
    midx = x[:, 0].astype(jnp.int32)
    gidx = x[:, 1].astype(jnp.int32)
    wb = jnp.stack([
        jnp.broadcast_to(fc_w.reshape(()), (16,)),
        jnp.broadcast_to(fc_b.reshape(()), (16,)),
    ]).astype(jnp.float32)
    out = _run(mT, g_table, midx, gidx, wb)
    return out.reshape(B, 1)


# speedup vs baseline: 24.2575x; 1.1705x over previous
import functools

import jax
import jax.numpy as jnp
from jax import lax
from jax.experimental import pallas as pl
from jax.experimental.pallas import tpu as pltpu
from jax.experimental.pallas import tpu_sc as plsc

B = 16384
DIM = 64
NW = 32
ROWS_PER_W = B // NW
GS = 4
GROUPS = ROWS_PER_W // GS
PAD = 16


def _body(mT_hbm, g_hbm, midx_hbm, gidx_hbm, perm_hbm, wb_hbm, out_hbm,
          midx_v, gidx_v, perm_v, win0_v, win1_v, gr0_v, gr1_v, res_v,
          tmp_v, wb_v, sem0, sem1):
    wid = lax.axis_index("s") * 2 + lax.axis_index("c")
    base = wid * ROWS_PER_W

    pltpu.sync_copy(midx_hbm.at[pl.ds(base, ROWS_PER_W)],
                    midx_v.at[pl.ds(0, ROWS_PER_W)])
    pltpu.sync_copy(gidx_hbm.at[pl.ds(base, ROWS_PER_W)],
                    gidx_v.at[pl.ds(0, ROWS_PER_W)])
    pltpu.sync_copy(perm_hbm.at[pl.ds(base, ROWS_PER_W)], perm_v)
    pltpu.sync_copy(wb_hbm, wb_v)

    w = wb_v[0, :]
    bb = wb_v[1, :]
    iota = lax.iota(jnp.int32, 16)
    samp16 = iota & 3
    jq = iota >> 2
    fold8 = iota ^ 8
    fold4 = iota ^ 4
    mask4 = iota < 4
    zero16 = jnp.zeros((16,), jnp.int32)
    one16 = jnp.ones((16,), jnp.int32)
    two16 = jnp.full((16,), 2, jnp.int32)

    def windows(g):
        mv = midx_v[pl.ds(g * GS, 16)]
        ws = [mv[k] >> 7 for k in range(GS)]
        news = [True] + [ws[k] != ws[k - 1] for k in range(1, GS)]
        return ws, news

    def fire(g, win_v, gr_v, sem):
        mv = midx_v[pl.ds(g * GS, 16)]
        gv = gidx_v[pl.ds(g * GS, 16)]
        ws, news = windows(g)
        for k in range(GS):
            woff = pl.multiple_of(ws[k] * 128, 128)
            if k == 0:
                pltpu.async_copy(mT_hbm.at[:, pl.ds(woff, 128)],
                                 win_v.at[k], sem)
            else:
                @pl.when(news[k])
                def _(woff=woff, k=k):
                    pltpu.async_copy(mT_hbm.at[:, pl.ds(woff, 128)],
                                     win_v.at[k], sem)
            pltpu.async_copy(g_hbm.at[pl.ds(gv[k], 1)],
                             gr_v.at[pl.ds(k, 1)], sem)

    def drain(g, win_v, gr_v, sem):
        _, news = windows(g)
        for k in range(GS):
            if k == 0:
                pltpu.make_async_copy(mT_hbm.at[:, pl.ds(0, 128)],
                                      win_v.at[k], sem).wait()
            else:
                @pl.when(news[k])
                def _(k=k):
                    pltpu.make_async_copy(mT_hbm.at[:, pl.ds(0, 128)],
                                          win_v.at[k], sem).wait()
        pltpu.make_async_copy(g_hbm.at[pl.ds(0, GS)], gr_v, sem).wait()

    def compute(g, win_v, gr_v):
        goff = g * GS
        ws, news = windows(g)
        slots = [jnp.int32(0)]
        for k in range(1, GS):
            slots.append(jnp.where(news[k], jnp.int32(k), slots[k - 1]))
        slotvec = jnp.where(samp16 == 0, slots[0],
                            jnp.where(samp16 == 1, slots[1],
                                      jnp.where(samp16 == 2, slots[2],
                                                slots[3])))
        lanevec = plsc.load_gather(midx_v, [goff + samp16]) & 127
        zero = jnp.zeros((16,), jnp.float32)
        dot = zero
        nm2 = zero
        ng2 = zero
        for jb in range(0, DIM, 4):
            jv = jq + jb
            mj = plsc.load_gather(win_v, [slotvec, jv, lanevec])
            gj = plsc.load_gather(gr_v, [samp16, jv])
            dot = dot + mj * gj
            nm2 = nm2 + mj * mj
            ng2 = ng2 + gj * gj
        for fold in (fold8, fold4):
            tmp_v[0, :] = dot
            tmp_v[1, :] = nm2
            tmp_v[2, :] = ng2
            dot = dot + plsc.load_gather(tmp_v, [zero16, fold])
            nm2 = nm2 + plsc.load_gather(tmp_v, [one16, fold])
            ng2 = ng2 + plsc.load_gather(tmp_v, [two16, fold])
        d = jnp.maximum(nm2, 1e-16) * jnp.maximum(ng2, 1e-16)
        di = plsc.bitcast(d, jnp.int32)
        y = plsc.bitcast(jnp.int32(0x5F3759DF) - (di >> 1), jnp.float32)
        for _ in range(3):
            y = y * (1.5 - 0.5 * d * y * y)
        cos = dot * y
        z = cos * w + bb
        sig = 1.0 / (1.0 + jnp.exp(-z))
        plsc.store_compressed(res_v.at[pl.ds(goff, 16)], sig, mask=mask4)

    fire(0, win0_v, gr0_v, sem0)

    def pair(g2, _):
        g0 = g2 * 2
        fire(g0 + 1, win1_v, gr1_v, sem1)
        drain(g0, win0_v, gr0_v, sem0)
        compute(g0, win0_v, gr0_v)

        @pl.when(g0 + 2 < GROUPS)
        def _():
            fire(g0 + 2, win0_v, gr0_v, sem0)

        drain(g0 + 1, win1_v, gr1_v, sem1)
        compute(g0 + 1, win1_v, gr1_v)
        return 0

    lax.fori_loop(0, GROUPS // 2, pair, 0)

    pltpu.async_copy(res_v.at[pl.ds(0, ROWS_PER_W)],
                     out_hbm.at[perm_v], sem0)
    pltpu.make_async_copy(res_v.at[pl.ds(0, ROWS_PER_W)],
                          out_hbm.at[pl.ds(0, ROWS_PER_W)], sem0).wait()


@jax.jit
def _run(mT, g_table, midx, gidx, perm, wb):
    mesh = plsc.VectorSubcoreMesh(core_axis_name="c", subcore_axis_name="s")
    f = functools.partial(
        pl.kernel,
        mesh=mesh,
        out_type=jax.ShapeDtypeStruct((B,), jnp.float32),
        scratch_types=[
            pltpu.VMEM((ROWS_PER_W + PAD,), jnp.int32),
            pltpu.VMEM((ROWS_PER_W + PAD,), jnp.int32),
            pltpu.VMEM((ROWS_PER_W,), jnp.int32),
            pltpu.VMEM((GS, DIM, 128), jnp.float32),
            pltpu.VMEM((GS, DIM, 128), jnp.float32),
            pltpu.VMEM((GS, DIM), jnp.float32),
            pltpu.VMEM((GS, DIM), jnp.float32),
            pltpu.VMEM((ROWS_PER_W + PAD,), jnp.float32),
            pltpu.VMEM((3, 16), jnp.float32),
            pltpu.VMEM((2, 16), jnp.float32),
            pltpu.SemaphoreType.DMA,
            pltpu.SemaphoreType.DMA,
        ],
        compiler_params=pltpu.CompilerParams(needs_layout_passes=False),
    )(_body)
    return f(mT, g_table, midx, gidx, perm, wb)


def kernel(x, m_table, g_table, fc_w, fc_b):
    mT = m_table.T
    midx = x[:, 0].astype(jnp.int32)
    gidx = x[:, 1].astype(jnp.int32)
    order = jnp.argsort(midx).astype(jnp.int32)
    midx_s = jnp.take(midx, order)
    gidx_s = jnp.take(gidx, order)
    wb = jnp.stack([
        jnp.broadcast_to(fc_w.reshape(()), (16,)),
        jnp.broadcast_to(fc_b.reshape(()), (16,)),
    ]).astype(jnp.float32)
    out = _run(mT, g_table, midx_s, gidx_s, order, wb)
    return out.reshape(B, 1)


# speedup vs baseline: 25.6752x; 1.0584x over previous
import functools

import jax
import jax.numpy as jnp
from jax import lax
from jax.experimental import pallas as pl
from jax.experimental.pallas import tpu as pltpu
from jax.experimental.pallas import tpu_sc as plsc

B = 16384
DIM = 64
NW = 32
ROWS_PER_W = B // NW
GS = 4
GROUPS = ROWS_PER_W // GS
PAD = 16


def _body(mT_hbm, g_hbm, midx_hbm, gidx_hbm, perm_hbm, wb_hbm, out_hbm,
          midx_v, gidx_v, perm_v, win0_v, win1_v, gr0_v, gr1_v, res_v,
          tmp_v, wb_v, sem0, sem1):
    wid = lax.axis_index("s") * 2 + lax.axis_index("c")
    base = wid * ROWS_PER_W

    pltpu.sync_copy(perm_hbm.at[pl.ds(base, ROWS_PER_W)], perm_v)
    pltpu.sync_copy(wb_hbm, wb_v)
    cmi = pltpu.async_copy(midx_hbm.at[perm_v],
                           midx_v.at[pl.ds(0, ROWS_PER_W)], sem0)
    cgi = pltpu.async_copy(gidx_hbm.at[perm_v],
                           gidx_v.at[pl.ds(0, ROWS_PER_W)], sem1)
    cmi.wait()
    cgi.wait()

    w = wb_v[0, :]
    bb = wb_v[1, :]
    iota = lax.iota(jnp.int32, 16)
    samp16 = iota & 3
    jq = iota >> 2
    fold8 = iota ^ 8
    fold4 = iota ^ 4
    mask4 = iota < 4
    zero16 = jnp.zeros((16,), jnp.int32)
    one16 = jnp.ones((16,), jnp.int32)
    two16 = jnp.full((16,), 2, jnp.int32)

    def windows(g):
        mv = midx_v[pl.ds(g * GS, 16)]
        ws = [mv[k] >> 7 for k in range(GS)]
        news = [True] + [ws[k] != ws[k - 1] for k in range(1, GS)]
        return ws, news

    def fire(g, win_v, gr_v, sem):
        mv = midx_v[pl.ds(g * GS, 16)]
        gv = gidx_v[pl.ds(g * GS, 16)]
        ws, news = windows(g)
        for k in range(GS):
            woff = pl.multiple_of(ws[k] * 128, 128)
            if k == 0:
                pltpu.async_copy(mT_hbm.at[:, pl.ds(woff, 128)],
                                 win_v.at[k], sem)
            else:
                @pl.when(news[k])
                def _(woff=woff, k=k):
                    pltpu.async_copy(mT_hbm.at[:, pl.ds(woff, 128)],
                                     win_v.at[k], sem)
            pltpu.async_copy(g_hbm.at[pl.ds(gv[k], 1)],
                             gr_v.at[pl.ds(k, 1)], sem)

    def drain(g, win_v, gr_v, sem):
        _, news = windows(g)
        for k in range(GS):
            if k == 0:
                pltpu.make_async_copy(mT_hbm.at[:, pl.ds(0, 128)],
                                      win_v.at[k], sem).wait()
            else:
                @pl.when(news[k])
                def _(k=k):
                    pltpu.make_async_copy(mT_hbm.at[:, pl.ds(0, 128)],
                                          win_v.at[k], sem).wait()
        pltpu.make_async_copy(g_hbm.at[pl.ds(0, GS)], gr_v, sem).wait()

    def compute(g, win_v, gr_v):
        goff = g * GS
        ws, news = windows(g)
        slots = [jnp.int32(0)]
        for k in range(1, GS):
            slots.append(jnp.where(news[k], jnp.int32(k), slots[k - 1]))
        slotvec = jnp.where(samp16 == 0, slots[0],
                            jnp.where(samp16 == 1, slots[1],
                                      jnp.where(samp16 == 2, slots[2],
                                                slots[3])))
        lanevec = plsc.load_gather(midx_v, [goff + samp16]) & 127
        zero = jnp.zeros((16,), jnp.float32)
        dot = zero
        nm2 = zero
        ng2 = zero
        for jb in range(0, DIM, 4):
            jv = jq + jb
            mj = plsc.load_gather(win_v, [slotvec, jv, lanevec])
            gj = plsc.load_gather(gr_v, [samp16, jv])
            dot = dot + mj * gj
            nm2 = nm2 + mj * mj
            ng2 = ng2 + gj * gj
        for fold in (fold8, fold4):
            tmp_v[0, :] = dot
            tmp_v[1, :] = nm2
            tmp_v[2, :] = ng2
            dot = dot + plsc.load_gather(tmp_v, [zero16, fold])
            nm2 = nm2 + plsc.load_gather(tmp_v, [one16, fold])
            ng2 = ng2 + plsc.load_gather(tmp_v, [two16, fold])
        d = jnp.maximum(nm2, 1e-16) * jnp.maximum(ng2, 1e-16)
        di = plsc.bitcast(d, jnp.int32)
        y = plsc.bitcast(jnp.int32(0x5F3759DF) - (di >> 1), jnp.float32)
        for _ in range(3):
            y = y * (1.5 - 0.5 * d * y * y)
        cos = dot * y
        z = cos * w + bb
        sig = 1.0 / (1.0 + jnp.exp(-z))
        plsc.store_compressed(res_v.at[pl.ds(goff, 16)], sig, mask=mask4)

    fire(0, win0_v, gr0_v, sem0)

    def pair(g2, _):
        g0 = g2 * 2
        fire(g0 + 1, win1_v, gr1_v, sem1)
        drain(g0, win0_v, gr0_v, sem0)
        compute(g0, win0_v, gr0_v)

        @pl.when(g0 + 2 < GROUPS)
        def _():
            fire(g0 + 2, win0_v, gr0_v, sem0)

        drain(g0 + 1, win1_v, gr1_v, sem1)
        compute(g0 + 1, win1_v, gr1_v)
        return 0

    lax.fori_loop(0, GROUPS // 2, pair, 0)

    pltpu.async_copy(res_v.at[pl.ds(0, ROWS_PER_W)],
                     out_hbm.at[perm_v], sem0)
    pltpu.make_async_copy(res_v.at[pl.ds(0, ROWS_PER_W)],
                          out_hbm.at[pl.ds(0, ROWS_PER_W)], sem0).wait()


@jax.jit
def _run(mT, g_table, midx, gidx, perm, wb):
    mesh = plsc.VectorSubcoreMesh(core_axis_name="c", subcore_axis_name="s")
    f = functools.partial(
        pl.kernel,
        mesh=mesh,
        out_type=jax.ShapeDtypeStruct((B,), jnp.float32),
        scratch_types=[
            pltpu.VMEM((ROWS_PER_W + PAD,), jnp.int32),
            pltpu.VMEM((ROWS_PER_W + PAD,), jnp.int32),
            pltpu.VMEM((ROWS_PER_W,), jnp.int32),
            pltpu.VMEM((GS, DIM, 128), jnp.float32),
            pltpu.VMEM((GS, DIM, 128), jnp.float32),
            pltpu.VMEM((GS, DIM), jnp.float32),
            pltpu.VMEM((GS, DIM), jnp.float32),
            pltpu.VMEM((ROWS_PER_W + PAD,), jnp.float32),
            pltpu.VMEM((3, 16), jnp.float32),
            pltpu.VMEM((2, 16), jnp.float32),
            pltpu.SemaphoreType.DMA,
            pltpu.SemaphoreType.DMA,
        ],
        compiler_params=pltpu.CompilerParams(needs_layout_passes=False),
    )(_body)
    return f(mT, g_table, midx, gidx, perm, wb)


def kernel(x, m_table, g_table, fc_w, fc_b):
    mT = m_table.T
    midx = x[:, 0].astype(jnp.int32)
    gidx = x[:, 1].astype(jnp.int32)
    order = jnp.argsort(midx).astype(jnp.int32)
    wb = jnp.stack([
        jnp.broadcast_to(fc_w.reshape(()), (16,)),
        jnp.broadcast_to(fc_b.reshape(()), (16,)),
    ]).astype(jnp.float32)
    out = _run(mT, g_table, midx, gidx, order, wb)
    return out.reshape(B, 1)
